# baseline (speedup 1.0000x reference)
import math

import jax
import jax.numpy as jnp
from jax.experimental import pallas as pl
from jax.experimental.pallas import tpu as pltpu

K = 1000
ETA = 0.7
_T_TABLE = jnp.asarray(
    [(1.0 - ETA) * math.cos(2.0 * math.pi * i / K) + ETA for i in range(K)],
    dtype=jnp.float32,
)


def _body(steps_ref, table_ref, x_ref, o_ref):
    b = pl.program_id(0)
    idx = steps_ref[b] - 1
    idx = jnp.where(idx < 0, idx + K, idx)
    f = table_ref[idx]
    o_ref[...] = jnp.clip(x_ref[...] * f, 0.0, 1.0)


def kernel(x, fwd_steps):
    B, C, H, W = x.shape
    grid = (B, C)
    spec = pltpu.PrefetchScalarGridSpec(
        num_scalar_prefetch=2,
        grid=grid,
        in_specs=[
            pl.BlockSpec((1, 1, H, W), lambda b, c, *_: (b, c, 0, 0)),
        ],
        out_specs=pl.BlockSpec((1, 1, H, W), lambda b, c, *_: (b, c, 0, 0)),
    )
    return pl.pallas_call(
        _body,
        grid_spec=spec,
        out_shape=jax.ShapeDtypeStruct(x.shape, x.dtype),
    )(fwd_steps.astype(jnp.int32), _T_TABLE, x)


# speedup vs baseline: 1.4975x; 1.4975x over previous
import math

import jax
import jax.numpy as jnp
from jax.experimental import pallas as pl
from jax.experimental.pallas import tpu as pltpu

K = 1000
ETA = 0.7
_T_TABLE = jnp.asarray(
    [(1.0 - ETA) * math.cos(2.0 * math.pi * i / K) + ETA for i in range(K)],
    dtype=jnp.float32,
)


def _body(steps_ref, table_ref, x_ref, o_ref):
    b = pl.program_id(0)
    idx = steps_ref[b] - 1
    idx = jnp.where(idx < 0, idx + K, idx)
    f = table_ref[idx]
    o_ref[...] = jnp.clip(x_ref[...] * f, 0.0, 1.0)


def kernel(x, fwd_steps):
    B, C, H, W = x.shape
    x2 = x.reshape(B, C * H, W)
    grid = (B,)
    spec = pltpu.PrefetchScalarGridSpec(
        num_scalar_prefetch=2,
        grid=grid,
        in_specs=[
            pl.BlockSpec((1, C * H, W), lambda b, *_: (b, 0, 0)),
        ],
        out_specs=pl.BlockSpec((1, C * H, W), lambda b, *_: (b, 0, 0)),
    )
    out = pl.pallas_call(
        _body,
        grid_spec=spec,
        out_shape=jax.ShapeDtypeStruct(x2.shape, x2.dtype),
    )(fwd_steps.astype(jnp.int32), _T_TABLE, x2)
    return out.reshape(B, C, H, W)


# speedup vs baseline: 1.5283x; 1.0206x over previous
import math

import jax
import jax.numpy as jnp
from jax.experimental import pallas as pl
from jax.experimental.pallas import tpu as pltpu

K = 1000
ETA = 0.7
_T_TABLE = jnp.asarray(
    [(1.0 - ETA) * math.cos(2.0 * math.pi * i / K) + ETA for i in range(K)],
    dtype=jnp.float32,
)


_ROWS = 2


def _body(steps_ref, table_ref, x_ref, o_ref):
    g = pl.program_id(0)
    fs = []
    for r in range(_ROWS):
        idx = steps_ref[g * _ROWS + r] - 1
        idx = jnp.where(idx < 0, idx + K, idx)
        fs.append(table_ref[idx])
    factors = jnp.stack(fs).reshape(_ROWS, 1, 1)
    o_ref[...] = jnp.clip(x_ref[...] * factors, 0.0, 1.0)


def kernel(x, fwd_steps):
    B, C, H, W = x.shape
    x2 = x.reshape(B, C * H, W)
    grid = (B // _ROWS,)
    spec = pltpu.PrefetchScalarGridSpec(
        num_scalar_prefetch=2,
        grid=grid,
        in_specs=[
            pl.BlockSpec((_ROWS, C * H, W), lambda b, *_: (b, 0, 0)),
        ],
        out_specs=pl.BlockSpec((_ROWS, C * H, W), lambda b, *_: (b, 0, 0)),
    )
    out = pl.pallas_call(
        _body,
        grid_spec=spec,
        out_shape=jax.ShapeDtypeStruct(x2.shape, x2.dtype),
        compiler_params=pltpu.CompilerParams(
            dimension_semantics=("parallel",),
        ),
    )(fwd_steps.astype(jnp.int32), _T_TABLE, x2)
    return out.reshape(B, C, H, W)
